# baseline (speedup 1.0000x reference)
import jax
import jax.numpy as jnp
from jax.experimental import pallas as pl
from jax.experimental.pallas import tpu as pltpu

_DIM = 512
_N_BUCKETS = 1024
_BANDWIDTH = 4.0
_N_HASHES = 128
_KSPLIT = 2
_KBLOCK = _DIM // _KSPLIT


def _lsh_block_kernel(x_ref, rv_ref, out_ref):
    k = pl.program_id(1)
    partial = jnp.dot(x_ref[...], rv_ref[...], preferred_element_type=jnp.float32)

    @pl.when(k == 0)
    def _():
        out_ref[...] = partial

    @pl.when(k == _KSPLIT - 1)
    def _():
        proj = out_ref[...] + partial
        buckets = jnp.floor(proj * (1.0 / _BANDWIDTH)).astype(jnp.int32)
        out_ref[...] = (buckets & (_N_BUCKETS - 1)).astype(jnp.float32)


def kernel(x, random_vectors):
    n = x.shape[0]
    block = 16392
    grid_r = (n + block - 1) // block
    return pl.pallas_call(
        _lsh_block_kernel,
        grid=(grid_r, _KSPLIT),
        in_specs=[
            pl.BlockSpec((block, _KBLOCK), lambda i, k: (i, k)),
            pl.BlockSpec((_KBLOCK, _N_HASHES), lambda i, k: (k, 0)),
        ],
        out_specs=pl.BlockSpec((block, _N_HASHES), lambda i, k: (i, 0)),
        out_shape=jax.ShapeDtypeStruct((n, _N_HASHES), jnp.float32),
        compiler_params=pltpu.CompilerParams(
            dimension_semantics=("parallel", "arbitrary"),
            vmem_limit_bytes=120 * 1024 * 1024,
        ),
    )(x, random_vectors)


# speedup vs baseline: 1.0743x; 1.0743x over previous
import jax
import jax.numpy as jnp
from jax.experimental import pallas as pl
from jax.experimental.pallas import tpu as pltpu

_DIM = 512
_N_BUCKETS = 1024
_BANDWIDTH = 4.0
_N_HASHES = 128


def _lsh_block_kernel(x_ref, rv_ref, out_ref):
    proj = jnp.dot(x_ref[...], rv_ref[...], preferred_element_type=jnp.float32)
    buckets = jnp.floor(proj * (1.0 / _BANDWIDTH)).astype(jnp.int32)
    out_ref[...] = (buckets & (_N_BUCKETS - 1)).astype(jnp.float32)


def kernel(x, random_vectors):
    n = x.shape[0]
    block = 10928
    grid = ((n + block - 1) // block,)
    return pl.pallas_call(
        _lsh_block_kernel,
        grid=grid,
        in_specs=[
            pl.BlockSpec((block, _DIM), lambda i: (i, 0)),
            pl.BlockSpec((_DIM, _N_HASHES), lambda i: (0, 0)),
        ],
        out_specs=pl.BlockSpec((block, _N_HASHES), lambda i: (i, 0)),
        out_shape=jax.ShapeDtypeStruct((n, _N_HASHES), jnp.float32),
        compiler_params=pltpu.CompilerParams(
            dimension_semantics=("parallel",),
        ),
    )(x, random_vectors)


# speedup vs baseline: 1.0959x; 1.0201x over previous
import jax
import jax.numpy as jnp
from jax.experimental import pallas as pl
from jax.experimental.pallas import tpu as pltpu

_DIM = 512
_N_BUCKETS = 1024
_BANDWIDTH = 4.0
_N_HASHES = 128


def _lsh_block_kernel(x_ref, rv_ref, out_ref):
    proj = jnp.dot(x_ref[...], rv_ref[...], preferred_element_type=jnp.float32)
    buckets = jnp.floor(proj * (1.0 / _BANDWIDTH)).astype(jnp.int32)
    out_ref[...] = (buckets & (_N_BUCKETS - 1)).astype(jnp.float32)


def kernel(x, random_vectors):
    n = x.shape[0]
    block = 4096
    return pl.pallas_call(
        _lsh_block_kernel,
        grid=(n // block,),
        in_specs=[
            pl.BlockSpec((block, _DIM), lambda i: (i, 0)),
            pl.BlockSpec((_DIM, _N_HASHES), lambda i: (0, 0)),
        ],
        out_specs=pl.BlockSpec((block, _N_HASHES), lambda i: (i, 0)),
        out_shape=jax.ShapeDtypeStruct((n, _N_HASHES), jnp.float32),
        compiler_params=pltpu.CompilerParams(
            dimension_semantics=("parallel",),
        ),
    )(x, random_vectors)
